# baseline (speedup 1.0000x reference)
import functools
import math

import jax
import jax.numpy as jnp
from jax.experimental import pallas as pl
from jax.experimental.pallas import tpu as pltpu
from jax.experimental.pallas import tpu_sc as plsc

N_ROWS = 1024
N_COLS = 100000
KC = 1000
NCHUNK = N_COLS // KC
NBUF = 8
S0_SCALE = math.sqrt(2.0) * math.log(N_COLS - 1)
BETA = 1.0


def _stream(hbm_ref, buf, sem, body):
    def issue(c, slot):
        pltpu.make_async_copy(
            hbm_ref.at[pl.ds(c * KC, KC), :], buf.at[slot], sem.at[slot]
        ).start()

    for s in range(NBUF):
        issue(s, s)

    def outer(g, _):
        for s in range(NBUF):
            c = g * NBUF + s
            pltpu.make_async_copy(
                hbm_ref.at[pl.ds(c * KC, KC), :], buf.at[s], sem.at[s]
            ).wait()
            body(c, buf.at[s])

            @pl.when(c + NBUF < NCHUNK)
            def _():
                issue(c + NBUF, s)

        return 0

    n_groups = (NCHUNK + NBUF - 1) // NBUF
    if NCHUNK % NBUF == 0:
        jax.lax.fori_loop(0, n_groups, outer, 0)
    else:
        jax.lax.fori_loop(0, n_groups - 1, outer, 0)
        g = n_groups - 1
        for s in range(NCHUNK - (n_groups - 1) * NBUF):
            c = g * NBUF + s
            pltpu.make_async_copy(
                hbm_ref.at[pl.ds(c * KC, KC), :], buf.at[s], sem.at[s]
            ).wait()
            body(c, buf.at[s])


def _sc_gather_t(logits, labels):
    info = plsc.get_sparse_core_info()
    NC, NS = info.num_cores, info.num_subcores
    NW = NC * NS
    BPW = N_ROWS // NW
    xt = logits.T
    table = jnp.reshape(
        jnp.transpose(
            jnp.reshape(xt, (N_COLS // 8, 8, N_ROWS // 128, 128)),
            (0, 2, 1, 3),
        ),
        (N_COLS * N_ROWS // 128, 128),
    )
    mesh = plsc.VectorSubcoreMesh(core_axis_name="c", subcore_axis_name="s")

    @functools.partial(
        pl.kernel,
        mesh=mesh,
        out_type=jax.ShapeDtypeStruct((N_ROWS,), jnp.float32),
        compiler_params=pltpu.CompilerParams(needs_layout_passes=False),
        scratch_types=[
            pltpu.VMEM((BPW,), jnp.int32),
            pltpu.VMEM((BPW,), jnp.int32),
            pltpu.VMEM((BPW, 128), jnp.float32),
            pltpu.VMEM((BPW,), jnp.float32),
            pltpu.SemaphoreType.DMA,
        ],
    )
    def k(table_hbm, lab_hbm, out_hbm, lab_v, idx_v, rows_v, out_v, sem):
        wid = jax.lax.axis_index("s") * NC + jax.lax.axis_index("c")
        base = wid * BPW
        pltpu.sync_copy(lab_hbm.at[pl.ds(base, BPW)], lab_v)
        iota = jax.lax.iota(jnp.int32, 16)
        for g in range(BPW // 16):
            lab = lab_v[pl.ds(g * 16, 16)]
            j = base + g * 16 + iota
            row = (
                (jax.lax.shift_right_logical(lab, 3) * 64)
                + (jax.lax.shift_right_logical(j, 7) * 8)
                + jax.lax.bitwise_and(lab, 7)
            )
            idx_v[pl.ds(g * 16, 16)] = row
        pltpu.async_copy(table_hbm.at[idx_v], rows_v, sem).wait()
        for g in range(BPW // 16):
            j = base + g * 16 + iota
            vals = plsc.load_gather(
                rows_v, [g * 16 + iota, jax.lax.bitwise_and(j, 127)]
            )
            out_v[pl.ds(g * 16, 16)] = vals
        pltpu.sync_copy(out_v, out_hbm.at[pl.ds(base, BPW)])

    return k(table, labels.astype(jnp.int32))


def _pass1_kernel(hbm_ref, s0_ref, buf, sem):
    s0_ref[...] = jnp.zeros_like(s0_ref)

    def body(c, chunk):
        x = chunk[...]
        e0 = jnp.exp(S0_SCALE * x)
        s0_ref[...] += jnp.sum(e0, axis=0, keepdims=True)

    _stream(hbm_ref, buf, sem, body)


def _pass2_kernel(s_ref, hbm_ref, s1_ref, ss_ref, buf, sem):
    s = s_ref[0, 0]
    s1_ref[...] = jnp.zeros_like(s1_ref)
    ss_ref[...] = jnp.zeros_like(ss_ref)

    def body(c, chunk):
        x = chunk[...]
        s1_ref[...] += jnp.sum(jnp.exp(x), axis=0, keepdims=True)
        ss_ref[...] += jnp.sum(jnp.exp(s * x), axis=0, keepdims=True)

    _stream(hbm_ref, buf, sem, body)


def _acos(x):
    ax = jnp.abs(x)
    p = jnp.float32(-0.0012624911)
    for c in (0.0066700901, -0.0170881256, 0.0308918810, -0.0501743046,
              0.0889789874, -0.2145988016, 1.5707963050):
        p = p * ax + jnp.float32(c)
    r = jnp.sqrt(jnp.maximum(0.0, 1.0 - ax)) * p
    return jnp.where(x >= 0.0, r, jnp.float32(math.pi) - r)


def _two_kth_smallest(c, ka, kb, n_iter=48):
    def body(_, carry):
        lo_a, hi_a, lo_b, hi_b = carry
        mid_a = 0.5 * (lo_a + hi_a)
        mid_b = 0.5 * (lo_b + hi_b)
        cnt_a = jnp.sum((c <= mid_a).astype(jnp.float32))
        cnt_b = jnp.sum((c <= mid_b).astype(jnp.float32))
        ta = cnt_a >= (ka + 1)
        tb = cnt_b >= (kb + 1)
        return (
            jnp.where(ta, lo_a, mid_a), jnp.where(ta, mid_a, hi_a),
            jnp.where(tb, lo_b, mid_b), jnp.where(tb, mid_b, hi_b),
        )

    init = (jnp.float32(-1.1), jnp.float32(1.1),
            jnp.float32(-1.1), jnp.float32(1.1))
    _, hi_a, _, hi_b = jax.lax.fori_loop(0, n_iter, body, init)
    return hi_a, hi_b


def _mid_kernel(s0_ref, t_ref, s_out, mt_out):
    t = t_ref[...]
    sum0 = jnp.sum(s0_ref[...]) - jnp.sum(jnp.exp(S0_SCALE * t))
    b_avg = sum0 / N_ROWS
    c = jnp.clip(t, -1.0 + 1e-07, 1.0 - 1e-07)
    ca, cb = _two_kth_smallest(c, N_ROWS // 2 - 1, N_ROWS // 2)
    theta_med = 0.5 * (_acos(ca) + _acos(cb))
    cos_sum = ca * cb - jnp.sqrt(
        jnp.maximum(0.0, (1.0 - ca * ca)) * jnp.maximum(0.0, (1.0 - cb * cb))
    )
    cos_med = jnp.sqrt(jnp.maximum(0.0, 0.5 * (1.0 + cos_sum)))
    denom = jnp.where(
        theta_med < jnp.float32(math.pi / 4.0),
        cos_med,
        jnp.float32(math.cos(math.pi / 4.0)),
    )
    s = jnp.log(b_avg) / denom
    s_out[...] = jnp.reshape(s, (1, 1))
    mt_out[...] = jnp.reshape(jnp.mean(t), (1, 1))


def _final_kernel(s1_ref, ss_ref, t_ref, s_ref, out_ref):
    t = t_ref[...]
    s = s_ref[...]
    loss1 = jnp.mean(jnp.log(ss_ref[...])) - s * jnp.mean(t)
    loss2 = jnp.mean(jnp.log(s1_ref[...])) - jnp.mean(t)
    out_ref[...] = (BETA * loss1 + loss2) / (1.0 + BETA)


def kernel(logits, labels):
    xt = logits.T

    rowvec = jax.ShapeDtypeStruct((1, N_ROWS), jnp.float32)
    scalar = jax.ShapeDtypeStruct((1, 1), jnp.float32)
    scratch = [
        pltpu.VMEM((NBUF, KC, N_ROWS), jnp.float32),
        pltpu.SemaphoreType.DMA((NBUF,)),
    ]

    t_rows = _sc_gather_t(logits, labels).reshape(1, N_ROWS)

    s0_rows = pl.pallas_call(
        _pass1_kernel,
        in_specs=[pl.BlockSpec(memory_space=pl.ANY)],
        out_specs=pl.BlockSpec(memory_space=pltpu.VMEM),
        out_shape=rowvec,
        scratch_shapes=scratch,
    )(xt)

    s_sc, mt_sc = pl.pallas_call(
        _mid_kernel,
        out_shape=[scalar, scalar],
    )(s0_rows, t_rows)
    del mt_sc

    s1_rows, ss_rows = pl.pallas_call(
        _pass2_kernel,
        in_specs=[
            pl.BlockSpec(memory_space=pltpu.SMEM),
            pl.BlockSpec(memory_space=pl.ANY),
        ],
        out_specs=[pl.BlockSpec(memory_space=pltpu.VMEM)] * 2,
        out_shape=[rowvec, rowvec],
        scratch_shapes=scratch,
    )(s_sc, xt)

    loss = pl.pallas_call(
        _final_kernel,
        out_shape=scalar,
    )(s1_rows, ss_rows, t_rows, s_sc)

    return loss[0, 0]


# speedup vs baseline: 1.0763x; 1.0763x over previous
import functools
import math

import jax
import jax.numpy as jnp
from jax.experimental import pallas as pl
from jax.experimental.pallas import tpu as pltpu
from jax.experimental.pallas import tpu_sc as plsc

N_ROWS = 1024
N_COLS = 100000
KC = 400
NCHUNK = N_COLS // KC
NBUF = 16
S0_SCALE = math.sqrt(2.0) * math.log(N_COLS - 1)
BETA = 1.0


def _stream(hbm_ref, buf, sem, body, after_prefetch=None):
    def issue(c, slot):
        pltpu.make_async_copy(
            hbm_ref.at[pl.ds(c * KC, KC), :], buf.at[slot], sem.at[slot]
        ).start()

    for s in range(NBUF):
        issue(s, s)

    hook_result = after_prefetch() if after_prefetch is not None else None

    def step(c, s):
        pltpu.make_async_copy(
            hbm_ref.at[pl.ds(c * KC, KC), :], buf.at[s], sem.at[s]
        ).wait()
        body(c, buf.at[s], hook_result)

        @pl.when(c + NBUF < NCHUNK)
        def _():
            issue(c + NBUF, s)

    def outer(g, _):
        for s in range(NBUF):
            step(g * NBUF + s, s)
        return 0

    n_groups = NCHUNK // NBUF
    jax.lax.fori_loop(0, n_groups, outer, 0)
    for s in range(NCHUNK - n_groups * NBUF):
        step(n_groups * NBUF + s, s)
    return hook_result


def _sc_gather_t(logits, labels):
    info = plsc.get_sparse_core_info()
    NC, NS = info.num_cores, info.num_subcores
    NW = NC * NS
    BPW = N_ROWS // NW
    xt = logits.T
    table = jnp.reshape(
        jnp.transpose(
            jnp.reshape(xt, (N_COLS // 8, 8, N_ROWS // 128, 128)),
            (0, 2, 1, 3),
        ),
        (N_COLS * N_ROWS // 128, 128),
    )
    mesh = plsc.VectorSubcoreMesh(core_axis_name="c", subcore_axis_name="s")

    @functools.partial(
        pl.kernel,
        mesh=mesh,
        out_type=jax.ShapeDtypeStruct((N_ROWS,), jnp.float32),
        compiler_params=pltpu.CompilerParams(needs_layout_passes=False),
        scratch_types=[
            pltpu.VMEM((BPW,), jnp.int32),
            pltpu.VMEM((BPW,), jnp.int32),
            pltpu.VMEM((BPW, 128), jnp.float32),
            pltpu.VMEM((BPW,), jnp.float32),
            pltpu.SemaphoreType.DMA,
        ],
    )
    def k(table_hbm, lab_hbm, out_hbm, lab_v, idx_v, rows_v, out_v, sem):
        wid = jax.lax.axis_index("s") * NC + jax.lax.axis_index("c")
        base = wid * BPW
        pltpu.sync_copy(lab_hbm.at[pl.ds(base, BPW)], lab_v)
        iota = jax.lax.iota(jnp.int32, 16)
        for g in range(BPW // 16):
            lab = lab_v[pl.ds(g * 16, 16)]
            j = base + g * 16 + iota
            row = (
                (jax.lax.shift_right_logical(lab, 3) * 64)
                + (jax.lax.shift_right_logical(j, 7) * 8)
                + jax.lax.bitwise_and(lab, 7)
            )
            idx_v[pl.ds(g * 16, 16)] = row
        pltpu.async_copy(table_hbm.at[idx_v], rows_v, sem).wait()
        for g in range(BPW // 16):
            j = base + g * 16 + iota
            vals = plsc.load_gather(
                rows_v, [g * 16 + iota, jax.lax.bitwise_and(j, 127)]
            )
            out_v[pl.ds(g * 16, 16)] = vals
        pltpu.sync_copy(out_v, out_hbm.at[pl.ds(base, BPW)])

    return k(table, labels.astype(jnp.int32))


def _pass1_kernel(hbm_ref, s0_ref, buf, sem):
    s0_ref[...] = jnp.zeros_like(s0_ref)

    def body(c, chunk, _):
        x = chunk[...]
        s0_ref[...] += jnp.sum(jnp.exp(S0_SCALE * x), axis=0, keepdims=True)

    _stream(hbm_ref, buf, sem, body)


def _acos(x):
    ax = jnp.abs(x)
    p = jnp.float32(-0.0012624911)
    for c in (0.0066700901, -0.0170881256, 0.0308918810, -0.0501743046,
              0.0889789874, -0.2145988016, 1.5707963050):
        p = p * ax + jnp.float32(c)
    r = jnp.sqrt(jnp.maximum(0.0, 1.0 - ax)) * p
    return jnp.where(x >= 0.0, r, jnp.float32(math.pi) - r)


def _two_kth_smallest(c, ka, kb, n_iter=48):
    def body(_, carry):
        lo_a, hi_a, lo_b, hi_b = carry
        mid_a = 0.5 * (lo_a + hi_a)
        mid_b = 0.5 * (lo_b + hi_b)
        cnt_a = jnp.sum((c <= mid_a).astype(jnp.float32))
        cnt_b = jnp.sum((c <= mid_b).astype(jnp.float32))
        ta = cnt_a >= (ka + 1)
        tb = cnt_b >= (kb + 1)
        return (
            jnp.where(ta, lo_a, mid_a), jnp.where(ta, mid_a, hi_a),
            jnp.where(tb, lo_b, mid_b), jnp.where(tb, mid_b, hi_b),
        )

    init = (jnp.float32(-1.1), jnp.float32(1.1),
            jnp.float32(-1.1), jnp.float32(1.1))
    _, hi_a, _, hi_b = jax.lax.fori_loop(0, n_iter, body, init)
    return hi_a, hi_b


def _adaptive_scale(s0, t):
    sum0 = jnp.sum(s0) - jnp.sum(jnp.exp(S0_SCALE * t))
    b_avg = sum0 / N_ROWS
    c = jnp.clip(t, -1.0 + 1e-07, 1.0 - 1e-07)
    ca, cb = _two_kth_smallest(c, N_ROWS // 2 - 1, N_ROWS // 2)
    theta_med = 0.5 * (_acos(ca) + _acos(cb))
    cos_sum = ca * cb - jnp.sqrt(
        jnp.maximum(0.0, (1.0 - ca * ca)) * jnp.maximum(0.0, (1.0 - cb * cb))
    )
    cos_med = jnp.sqrt(jnp.maximum(0.0, 0.5 * (1.0 + cos_sum)))
    denom = jnp.where(
        theta_med < jnp.float32(math.pi / 4.0),
        cos_med,
        jnp.float32(math.cos(math.pi / 4.0)),
    )
    return jnp.log(b_avg) / denom


def _pass2_kernel(s0_ref, t_ref, hbm_ref, out_ref, buf, sem, s1_acc, ss_acc):
    s1_acc[...] = jnp.zeros_like(s1_acc)
    ss_acc[...] = jnp.zeros_like(ss_acc)
    t = t_ref[...]

    def mid():
        return _adaptive_scale(s0_ref[...], t)

    def body(c, chunk, s):
        x = chunk[...]
        s1_acc[...] += jnp.sum(jnp.exp(x), axis=0, keepdims=True)
        ss_acc[...] += jnp.sum(jnp.exp(s * x), axis=0, keepdims=True)

    s = _stream(hbm_ref, buf, sem, body, after_prefetch=mid)

    loss1 = jnp.mean(jnp.log(ss_acc[...])) - s * jnp.mean(t)
    loss2 = jnp.mean(jnp.log(s1_acc[...])) - jnp.mean(t)
    out_ref[...] = jnp.reshape(
        (BETA * loss1 + loss2) / (1.0 + BETA), (1, 1)
    )


def kernel(logits, labels):
    xt = logits.T

    rowvec = jax.ShapeDtypeStruct((1, N_ROWS), jnp.float32)
    scalar = jax.ShapeDtypeStruct((1, 1), jnp.float32)
    scratch = [
        pltpu.VMEM((NBUF, KC, N_ROWS), jnp.float32),
        pltpu.SemaphoreType.DMA((NBUF,)),
    ]

    t_rows = _sc_gather_t(logits, labels).reshape(1, N_ROWS)

    s0_rows = pl.pallas_call(
        _pass1_kernel,
        in_specs=[pl.BlockSpec(memory_space=pl.ANY)],
        out_specs=pl.BlockSpec(memory_space=pltpu.VMEM),
        out_shape=rowvec,
        scratch_shapes=scratch,
    )(xt)

    loss = pl.pallas_call(
        _pass2_kernel,
        in_specs=[
            pl.BlockSpec(memory_space=pltpu.VMEM),
            pl.BlockSpec(memory_space=pltpu.VMEM),
            pl.BlockSpec(memory_space=pl.ANY),
        ],
        out_specs=pl.BlockSpec(memory_space=pltpu.VMEM),
        out_shape=scalar,
        scratch_shapes=scratch + [
            pltpu.VMEM((1, N_ROWS), jnp.float32),
            pltpu.VMEM((1, N_ROWS), jnp.float32),
        ],
    )(s0_rows, t_rows, xt)

    return loss[0, 0]
